# baseline (speedup 1.0000x reference)
import jax
import jax.numpy as jnp
from jax import lax
from jax.experimental import pallas as pl
from jax.experimental.pallas import tpu as pltpu
from jax.experimental.pallas import tpu_sc as plsc

B, K, C, NPOINT = 16, 16384, 256, 4096
NC, NS, L = 2, 16, 16
NW = NC * NS
WPB = NW // B
CH_W = C // WPB
PT_W = NPOINT // WPB
ROWS = 2


def _sc_gather_body(xyz_hbm, feat_hbm, idx_hbm, oxyz_hbm, ofeat_hbm,
                    idx_v, rows_v, fout_v, xyz_v, xout_v):
    cid = lax.axis_index("c")
    sid = lax.axis_index("s")
    wid = sid * NC + cid
    b = wid // WPB
    half = wid % WPB
    ch0 = half * CH_W
    p0 = half * PT_W

    lane = jnp.arange(L, dtype=jnp.int32)

    pltpu.sync_copy(idx_hbm.at[b], idx_v)

    pltpu.sync_copy(xyz_hbm.at[b], xyz_v)

    def xyz_step(i, _):
        iv = idx_v[pl.ds(p0 + i * L, L)]
        dst0 = (lane + i * L) * 3
        for j in range(3):
            v = plsc.load_gather(xyz_v, [iv * 3 + j])
            plsc.store_scatter(xout_v, [dst0 + j], v)
        return 0

    lax.fori_loop(0, PT_W // L, xyz_step, 0)
    pltpu.sync_copy(xout_v, oxyz_hbm.at[b, pl.ds(p0 * 3, PT_W * 3)])

    def feat_blk(blk, _):
        ch = ch0 + blk * ROWS
        pltpu.sync_copy(feat_hbm.at[b, pl.ds(ch * K, ROWS * K)], rows_v)

        def gstep(i, _):
            iv = idx_v[pl.ds(i * L, L)]
            for r in range(ROWS):
                v = plsc.load_gather(rows_v, [iv + r * K])
                fout_v[pl.ds(r * NPOINT + i * L, L)] = v
            return 0

        lax.fori_loop(0, NPOINT // L, gstep, 0)
        pltpu.sync_copy(fout_v,
                        ofeat_hbm.at[b, pl.ds(ch * NPOINT, ROWS * NPOINT)])
        return 0

    lax.fori_loop(0, CH_W // ROWS, feat_blk, 0)


@jax.jit
def _sc_gather(xyz_flat, feat_flat, sample_inds):
    mesh = plsc.VectorSubcoreMesh(core_axis_name="c", subcore_axis_name="s",
                                  num_cores=NC, num_subcores=NS)
    return pl.kernel(
        _sc_gather_body,
        out_type=(
            jax.ShapeDtypeStruct((B, NPOINT * 3), jnp.float32),
            jax.ShapeDtypeStruct((B, C * NPOINT), jnp.float32),
        ),
        mesh=mesh,
        compiler_params=pltpu.CompilerParams(needs_layout_passes=False),
        scratch_types=[
            pltpu.VMEM((NPOINT,), jnp.int32),
            pltpu.VMEM((ROWS * K,), jnp.float32),
            pltpu.VMEM((ROWS * NPOINT,), jnp.float32),
            pltpu.VMEM((K * 3,), jnp.float32),
            pltpu.VMEM((PT_W * 3,), jnp.float32),
        ],
    )(xyz_flat, feat_flat, sample_inds)


def kernel(xyz, features, sample_inds):
    oxyz, ofeat = _sc_gather(xyz.reshape(B, K * 3),
                             features.reshape(B, C * K),
                             sample_inds)
    return (oxyz.reshape(B, NPOINT, 3), ofeat.reshape(B, C, NPOINT),
            sample_inds)


# speedup vs baseline: 1.2344x; 1.2344x over previous
import jax
import jax.numpy as jnp
from jax import lax
from jax.experimental import pallas as pl
from jax.experimental.pallas import tpu as pltpu
from jax.experimental.pallas import tpu_sc as plsc

B, K, C, NPOINT = 16, 16384, 256, 4096
NC, NS, L = 2, 16, 16
NW = NC * NS
WPB = NW // B
CH_W = C // WPB
PT_W = NPOINT // WPB
ROWS = 2
RK = ROWS * K
RN = ROWS * NPOINT
NBLK = CH_W // ROWS


def _sc_gather_body(xyz_hbm, feat_hbm, idx_hbm, oxyz_hbm, ofeat_hbm,
                    idx_v, rows_v, fout_v, xout_v,
                    sin0, sin1, sout0, sout1):
    cid = lax.axis_index("c")
    sid = lax.axis_index("s")
    wid = sid * NC + cid
    b = wid // WPB
    half = wid % WPB
    ch0 = half * CH_W
    p0 = half * PT_W

    lane = jnp.arange(L, dtype=jnp.int32)

    pltpu.sync_copy(idx_hbm.at[b], idx_v)

    pltpu.sync_copy(xyz_hbm.at[b], rows_v.at[pl.ds(0, K * 3)])

    def xyz_step(i, _):
        iv = idx_v[pl.ds(p0 + i * L, L)]
        dst0 = (lane + i * L) * 3
        for j in range(3):
            v = plsc.load_gather(rows_v, [iv * 3 + j])
            plsc.store_scatter(xout_v, [dst0 + j], v)
        return 0

    lax.fori_loop(0, PT_W // L, xyz_step, 0, unroll=4)
    pltpu.sync_copy(xout_v, oxyz_hbm.at[b, pl.ds(p0 * 3, PT_W * 3)])

    sins = (sin0, sin1)
    souts = (sout0, sout1)

    def in_desc(blk, s):
        ch = ch0 + blk * ROWS
        return pltpu.make_async_copy(feat_hbm.at[b, pl.ds(ch * K, RK)],
                                     rows_v.at[pl.ds(s * RK, RK)], sins[s])

    def out_desc(blk, s):
        ch = ch0 + blk * ROWS
        return pltpu.make_async_copy(fout_v.at[pl.ds(s * RN, RN)],
                                     ofeat_hbm.at[b, pl.ds(ch * NPOINT, RN)],
                                     souts[s])

    in_desc(0, 0).start()
    in_desc(1, 1).start()

    def outer(g, _):
        for s in range(2):
            blk = g * 2 + s
            in_desc(blk, s).wait()

            @pl.when(blk >= 2)
            def _():
                out_desc(blk - 2, s).wait()

            def gstep(i, _):
                iv = idx_v[pl.ds(i * L, L)]
                for r in range(ROWS):
                    v = plsc.load_gather(rows_v, [iv + (s * ROWS + r) * K])
                    fout_v[pl.ds((s * ROWS + r) * NPOINT + i * L, L)] = v
                return 0

            lax.fori_loop(0, NPOINT // L, gstep, 0, unroll=4)
            out_desc(blk, s).start()

            @pl.when(blk + 2 < NBLK)
            def _():
                in_desc(blk + 2, s).start()
        return 0

    lax.fori_loop(0, NBLK // 2, outer, 0)
    for s in range(2):
        out_desc(NBLK - 2 + s, s).wait()


@jax.jit
def _sc_gather(xyz_flat, feat_flat, sample_inds):
    mesh = plsc.VectorSubcoreMesh(core_axis_name="c", subcore_axis_name="s",
                                  num_cores=NC, num_subcores=NS)
    return pl.kernel(
        _sc_gather_body,
        out_type=(
            jax.ShapeDtypeStruct((B, NPOINT * 3), jnp.float32),
            jax.ShapeDtypeStruct((B, C * NPOINT), jnp.float32),
        ),
        mesh=mesh,
        compiler_params=pltpu.CompilerParams(needs_layout_passes=False),
        scratch_types=[
            pltpu.VMEM((NPOINT,), jnp.int32),
            pltpu.VMEM((2 * RK,), jnp.float32),
            pltpu.VMEM((2 * RN,), jnp.float32),
            pltpu.VMEM((PT_W * 3,), jnp.float32),
            pltpu.SemaphoreType.DMA,
            pltpu.SemaphoreType.DMA,
            pltpu.SemaphoreType.DMA,
            pltpu.SemaphoreType.DMA,
        ],
    )(xyz_flat, feat_flat, sample_inds)


def kernel(xyz, features, sample_inds):
    oxyz, ofeat = _sc_gather(xyz.reshape(B, K * 3),
                             features.reshape(B, C * K),
                             sample_inds)
    return (oxyz.reshape(B, NPOINT, 3), ofeat.reshape(B, C, NPOINT),
            sample_inds)


# speedup vs baseline: 2.5601x; 2.0739x over previous
import jax
import jax.numpy as jnp
from jax import lax
from jax.experimental import pallas as pl
from jax.experimental.pallas import tpu as pltpu
from jax.experimental.pallas import tpu_sc as plsc

B, K, C, NPOINT = 16, 16384, 256, 4096
NC, NS, L = 2, 16, 16
NW = NC * NS
WPB = NW // B
CH_W = C // WPB
PT_W = NPOINT // WPB
ROWS = 2
RK = ROWS * K
RN = ROWS * NPOINT
NBLK = CH_W // ROWS


def _sc_gather_body(xyz_hbm, feat_hbm, idx_hbm, oxyz_hbm, ofeat_hbm,
                    idx_v, rows_v, fout_v, xout_v,
                    sin0, sin1, sout0, sout1):
    cid = lax.axis_index("c")
    sid = lax.axis_index("s")
    wid = sid * NC + cid
    b = wid // WPB
    half = wid % WPB
    ch0 = half * CH_W
    p0 = half * PT_W

    lane = jnp.arange(L, dtype=jnp.int32)

    pltpu.sync_copy(idx_hbm.at[b], idx_v)

    for j in range(3):
        pltpu.sync_copy(xyz_hbm.at[j, b], rows_v.at[pl.ds(j * K, K)])

    def xyz_step(i, _):
        iv = idx_v[pl.ds(p0 + i * L, L)]
        for j in range(3):
            v = plsc.load_gather(rows_v, [iv + j * K])
            xout_v[pl.ds(j * PT_W + i * L, L)] = v
        return 0

    lax.fori_loop(0, PT_W // L, xyz_step, 0, unroll=4)
    for j in range(3):
        pltpu.sync_copy(xout_v.at[pl.ds(j * PT_W, PT_W)],
                        oxyz_hbm.at[j, b, pl.ds(p0, PT_W)])

    sins = (sin0, sin1)
    souts = (sout0, sout1)

    def in_descs(blk, s):
        ch = ch0 + blk * ROWS
        return [
            pltpu.make_async_copy(feat_hbm.at[b, ch + r],
                                  rows_v.at[pl.ds((s * ROWS + r) * K, K)],
                                  sins[s])
            for r in range(ROWS)
        ]

    def out_descs(blk, s):
        ch = ch0 + blk * ROWS
        return [
            pltpu.make_async_copy(fout_v.at[pl.ds((s * ROWS + r) * NPOINT,
                                                  NPOINT)],
                                  ofeat_hbm.at[b, ch + r],
                                  souts[s])
            for r in range(ROWS)
        ]

    for s in range(2):
        for d in in_descs(s, s):
            d.start()

    def outer(g, _):
        for s in range(2):
            blk = g * 2 + s
            for d in in_descs(blk, s):
                d.wait()

            @pl.when(blk >= 2)
            def _():
                for d in out_descs(blk - 2, s):
                    d.wait()

            def gstep(i, _):
                iv = idx_v[pl.ds(i * L, L)]
                for r in range(ROWS):
                    v = plsc.load_gather(rows_v, [iv + (s * ROWS + r) * K])
                    fout_v[pl.ds((s * ROWS + r) * NPOINT + i * L, L)] = v
                return 0

            lax.fori_loop(0, NPOINT // L, gstep, 0, unroll=4)
            for d in out_descs(blk, s):
                d.start()

            @pl.when(blk + 2 < NBLK)
            def _():
                for d in in_descs(blk + 2, s):
                    d.start()
        return 0

    lax.fori_loop(0, NBLK // 2, outer, 0)
    for s in range(2):
        for d in out_descs(NBLK - 2 + s, s):
            d.wait()


@jax.jit
def _sc_gather(xyz_t, features, sample_inds):
    mesh = plsc.VectorSubcoreMesh(core_axis_name="c", subcore_axis_name="s",
                                  num_cores=NC, num_subcores=NS)
    return pl.kernel(
        _sc_gather_body,
        out_type=(
            jax.ShapeDtypeStruct((3, B, NPOINT), jnp.float32),
            jax.ShapeDtypeStruct((B, C, NPOINT), jnp.float32),
        ),
        mesh=mesh,
        compiler_params=pltpu.CompilerParams(needs_layout_passes=False),
        scratch_types=[
            pltpu.VMEM((NPOINT,), jnp.int32),
            pltpu.VMEM((2 * RK,), jnp.float32),
            pltpu.VMEM((2 * RN,), jnp.float32),
            pltpu.VMEM((3 * PT_W,), jnp.float32),
            pltpu.SemaphoreType.DMA,
            pltpu.SemaphoreType.DMA,
            pltpu.SemaphoreType.DMA,
            pltpu.SemaphoreType.DMA,
        ],
    )(xyz_t, features, sample_inds)


def kernel(xyz, features, sample_inds):
    oxyz_t, ofeat = _sc_gather(jnp.transpose(xyz, (2, 0, 1)),
                               features, sample_inds)
    return (jnp.transpose(oxyz_t, (1, 2, 0)), ofeat, sample_inds)
